# baseline (speedup 1.0000x reference)
import jax
import jax.numpy as jnp
import numpy as np
from jax import lax
from jax.experimental import pallas as pl
from jax.experimental.pallas import tpu as pltpu
from jax.experimental.pallas import tpu_sc as plsc

B = 4096
NF = 26
VOCAB = 100000
EMB = 64
NV = NF + 1
TOTAL = B * NF
TOP_IN_PAD = 512

NC, NS = 2, 16
NW = NC * NS
PER_W = TOTAL // NW
ROWS_PER_W = PER_W // 128
CHUNK_ROWS = 13
NCHUNK = ROWS_PER_W // CHUNK_ROWS
CHUNK = CHUNK_ROWS * 128

BT = 256
GRID = B // BT


def _sc_gather_body(cat_hbm, table_hbm, out_hbm, cat_v, rows_v, sem):
    c = lax.axis_index("c")
    s = lax.axis_index("s")
    wid = s * NC + c
    for ch in range(NCHUNK):
        flat0 = wid * PER_W + ch * CHUNK
        pltpu.sync_copy(cat_hbm.at[pl.ds(flat0, CHUNK)], cat_v)

        def add_body(i, carry):
            p0 = i * 16
            off = ((p0 + lax.iota(jnp.int32, 16)) % NF) * VOCAB
            cat_v[pl.ds(p0, 16)] = cat_v[pl.ds(p0, 16)] + off
            return carry

        lax.fori_loop(0, CHUNK // 16, add_body, 0)

        copies = [
            pltpu.async_copy(
                table_hbm.at[cat_v.at[pl.ds(j * 128, 128)]],
                rows_v.at[pl.ds(j * 128, 128)],
                sem,
            )
            for j in range(CHUNK_ROWS)
        ]
        for cp in copies:
            cp.wait()
        out0 = wid * PER_W + ch * CHUNK
        pltpu.sync_copy(rows_v, out_hbm.at[pl.ds(out0, CHUNK)])


def _sc_gather(cat2, table):
    mesh = plsc.VectorSubcoreMesh(
        core_axis_name="c", subcore_axis_name="s", num_cores=NC, num_subcores=NS
    )
    return pl.kernel(
        _sc_gather_body,
        out_type=jax.ShapeDtypeStruct((TOTAL, EMB), jnp.float32),
        mesh=mesh,
        scratch_types=[
            pltpu.VMEM((CHUNK,), jnp.int32),
            pltpu.VMEM((CHUNK, EMB), jnp.float32),
            pltpu.SemaphoreType.DMA,
        ],
        compiler_params=pltpu.CompilerParams(use_tc_tiling_on_sc=False),
    )(cat2, table)


def _tc_body(num_ref, emb_ref, bw0, bb0, bw1, bb1, bw2, bb2,
             tw0, tb0, tw1, tb1, tw2, tb2, tw3, tb3, tw4, tb4,
             s_ref, out_ref):
    f32 = jnp.float32
    num = num_ref[...]
    h = jnp.maximum(jnp.dot(num, bw0[...], preferred_element_type=f32) + bb0[...], 0.0)
    h = jnp.maximum(jnp.dot(h, bw1[...], preferred_element_type=f32) + bb1[...], 0.0)
    bot = jnp.maximum(jnp.dot(h, bw2[...], preferred_element_type=f32) + bb2[...], 0.0)

    emb = emb_ref[...]
    t = jnp.concatenate([bot, emb], axis=1)
    s_mat = s_ref[...]

    parts = [bot]
    for i in range(1, NV):
        ti = t[:, i * EMB:(i + 1) * EMB]
        tin = jnp.concatenate([ti] * NV, axis=1)
        zi = jnp.dot(t * tin, s_mat, preferred_element_type=f32)
        parts.append(zi[:, :i])
    parts.append(jnp.zeros((BT, TOP_IN_PAD - EMB - NV * (NV - 1) // 2), f32))
    x = jnp.concatenate(parts, axis=1)

    x = jnp.maximum(jnp.dot(x, tw0[...], preferred_element_type=f32) + tb0[...], 0.0)
    x = jnp.maximum(jnp.dot(x, tw1[...], preferred_element_type=f32) + tb1[...], 0.0)
    x = jnp.maximum(jnp.dot(x, tw2[...], preferred_element_type=f32) + tb2[...], 0.0)
    x = jnp.maximum(jnp.dot(x, tw3[...], preferred_element_type=f32) + tb3[...], 0.0)
    out_ref[...] = jnp.dot(x, tw4[...], preferred_element_type=f32) + tb4[...]


def _full(shape):
    return pl.BlockSpec(shape, lambda i: (0,) * len(shape))


def _tc_forward(num, emb2, weights, s_mat):
    in_specs = [
        pl.BlockSpec((BT, num.shape[1]), lambda i: (i, 0)),
        pl.BlockSpec((BT, NF * EMB), lambda i: (i, 0)),
    ]
    for w in weights:
        in_specs.append(_full(w.shape))
    in_specs.append(_full(s_mat.shape))
    return pl.pallas_call(
        _tc_body,
        grid=(GRID,),
        in_specs=in_specs,
        out_specs=pl.BlockSpec((BT, 128), lambda i: (i, 0)),
        out_shape=jax.ShapeDtypeStruct((B, 128), jnp.float32),
    )(num, emb2, *weights, s_mat)


def kernel(numerical_input, categorical_inputs, emb_table,
           bw0, bb0, bw1, bb1, bw2, bb2,
           tw0, tb0, tw1, tb1, tw2, tb2, tw3, tb3, tw4, tb4):
    cat2 = categorical_inputs.reshape(TOTAL)
    emb_flat = _sc_gather(cat2, emb_table)
    emb2 = emb_flat.reshape(B, NF * EMB)

    s_mat = jnp.asarray(np.repeat(np.eye(NV, dtype=np.float32), EMB, axis=0))
    tw0p = jnp.pad(tw0, ((0, TOP_IN_PAD - tw0.shape[0]), (0, 0)))
    tw4p = jnp.pad(tw4, ((0, 0), (0, 127)))
    tb4p = jnp.pad(tb4.reshape(1, 1), ((0, 0), (0, 127)))
    weights = (bw0, bb0.reshape(1, -1), bw1, bb1.reshape(1, -1),
               bw2, bb2.reshape(1, -1),
               tw0p, tb0.reshape(1, -1), tw1, tb1.reshape(1, -1),
               tw2, tb2.reshape(1, -1), tw3, tb3.reshape(1, -1),
               tw4p, tb4p)
    out128 = _tc_forward(numerical_input, emb2, weights, s_mat)
    return out128[:, :1]
